# Initial kernel scaffold; baseline (speedup 1.0000x reference)
import functools

import jax
import jax.numpy as jnp
from jax import lax
from jax.experimental import pallas as pl
from jax.experimental.pallas import tpu as pltpu
from jax.experimental.pallas import tpu_sc as plsc

N = 10000
E = 320000
D = 128
DE = 16
NEG = 0.2

NC = 2
NS = 16
NW = NC * NS
EPW = E // NW
CE = 80
NCHUNK = EPW // CE
RPS = N // NS


def _z_body(er_ref, at_ref, sel_ref, z_ref):
    t = er_ref[...] * at_ref[...]
    l = lax.dot_general(t, sel_ref[...], (((1,), (0,)), ((), ())),
                        preferred_element_type=jnp.float32)
    l = jnp.where(l >= 0, l, NEG * l)
    z_ref[...] = jnp.exp(l)


def _edge_z(edge_r, attn_e):
    er2 = edge_r.reshape(E // 8, 8 * DE)
    at_tiled = jnp.tile(attn_e[:, 0], 8).reshape(1, 8 * DE)
    sel = jnp.kron(jnp.eye(8, dtype=jnp.float32),
                   jnp.ones((DE, 1), dtype=jnp.float32))
    B = 4000
    z8 = pl.pallas_call(
        _z_body,
        grid=(E // 8 // B,),
        in_specs=[pl.BlockSpec((B, 8 * DE), lambda i: (i, 0)),
                  pl.BlockSpec((1, 8 * DE), lambda i: (0, 0)),
                  pl.BlockSpec((8 * DE, 8), lambda i: (0, 0))],
        out_specs=pl.BlockSpec((B, 8), lambda i: (i, 0)),
        out_shape=jax.ShapeDtypeStruct((E // 8, 8), jnp.float32),
    )(er2, at_tiled, sel)
    return z8.reshape(E)


def _mm_body(f_ref, wn_ref, ws_ref, b_ref, fw_ref, sp_ref):
    f = f_ref[...]
    fw_ref[...] = lax.dot_general(f, wn_ref[...], (((1,), (1,)), ((), ())),
                                  preferred_element_type=jnp.float32)
    sp_ref[...] = lax.dot_general(f, ws_ref[...], (((1,), (1,)), ((), ())),
                                  preferred_element_type=jnp.float32) + b_ref[...]


def _pre_matmuls(feat, W_neigh, W_self, bias):
    B = 1000
    return pl.pallas_call(
        _mm_body,
        grid=(N // B,),
        in_specs=[pl.BlockSpec((B, D), lambda i: (i, 0)),
                  pl.BlockSpec((D, D), lambda i: (0, 0)),
                  pl.BlockSpec((D, D), lambda i: (0, 0)),
                  pl.BlockSpec((1, D), lambda i: (0, 0))],
        out_specs=[pl.BlockSpec((B, D), lambda i: (i, 0)),
                   pl.BlockSpec((B, D), lambda i: (i, 0))],
        out_shape=[jax.ShapeDtypeStruct((N, D), jnp.float32),
                   jax.ShapeDtypeStruct((N, D), jnp.float32)],
    )(feat, W_neigh, W_self, bias.reshape(1, D))


def _splat(vec16, j):
    idx = jnp.full((16,), j, dtype=jnp.int32)
    return lax.gather(
        vec16, idx[:, None],
        lax.GatherDimensionNumbers(offset_dims=(), collapsed_slice_dims=(0,),
                                   start_index_map=(0,)),
        slice_sizes=(1,), mode=lax.GatherScatterMode.PROMISE_IN_BOUNDS)


def _sc_body(fw_hbm, src_hbm, dst_hbm, z_hbm, U_hbm, s_hbm,
             src_v, dst_v, z_v, rows_v, s_loc, U_sh, sem):
    c = lax.axis_index("c")
    sid = lax.axis_index("s")
    wid = sid * NC + c
    zero16 = jnp.zeros((16,), jnp.float32)

    @pl.loop(0, N, step=16)
    def _(i):
        s_loc[pl.ds(i, 16)] = zero16

    @pl.loop(0, CE)
    def _(r):
        for v in range(D // 16):
            rows_v[r, pl.ds(v * 16, 16)] = zero16

    base_row = sid * RPS
    off = 0
    for cnt in (80, 80, 80, 80, 80, 80, 80, 65):
        pltpu.sync_copy(rows_v.at[pl.ds(0, cnt)],
                        U_sh.at[pl.ds(base_row + off, cnt)])
        off += cnt
    plsc.subcore_barrier()

    ebase = wid * EPW

    @pl.loop(0, NCHUNK)
    def _(k):
        eoff = ebase + k * CE
        pltpu.sync_copy(src_hbm.at[pl.ds(eoff, CE)], src_v)
        pltpu.sync_copy(dst_hbm.at[pl.ds(eoff, CE)], dst_v)
        pltpu.sync_copy(z_hbm.at[pl.ds(eoff, CE)], z_v)
        pltpu.async_copy(fw_hbm.at[src_v], rows_v, sem).wait()

        @pl.loop(0, CE // 16)
        def _(g):
            z16 = z_v[pl.ds(g * 16, 16)]
            d16 = dst_v[pl.ds(g * 16, 16)]
            plsc.addupdate_scatter(s_loc, [d16], z16)
            for j in range(16):
                zj = _splat(z16, j)
                e = g * 16 + j
                for v in range(D // 16):
                    sl = pl.ds(v * 16, 16)
                    rows_v[e, sl] = rows_v[e, sl] * zj

        pltpu.sync_copy(rows_v, U_sh.at[dst_v], add=True)

    plsc.subcore_barrier()
    pltpu.sync_copy(s_loc, s_hbm.at[wid])
    off = 0
    for cnt in (80, 80, 80, 80, 80, 80, 80, 65):
        pltpu.sync_copy(U_sh.at[pl.ds(base_row + off, cnt)],
                        U_hbm.at[c, pl.ds(base_row + off, cnt)])
        off += cnt


def _sc_aggregate(fw, src, dst, z):
    mesh = plsc.VectorSubcoreMesh(core_axis_name="c", subcore_axis_name="s")
    kern = pl.kernel(
        _sc_body,
        out_type=[jax.ShapeDtypeStruct((NC, N, D), jnp.float32),
                  jax.ShapeDtypeStruct((NW, N), jnp.float32)],
        mesh=mesh,
        scratch_types=[
            pltpu.VMEM((CE,), jnp.int32),
            pltpu.VMEM((CE,), jnp.int32),
            pltpu.VMEM((CE,), jnp.float32),
            pltpu.VMEM((CE, D), jnp.float32),
            pltpu.VMEM((N,), jnp.float32),
            pltpu.VMEM_SHARED((N, D), jnp.float32),
            pltpu.SemaphoreType.DMA,
        ])
    return kern(fw, src, dst, z)


def _fin_body(sp_ref, U_ref, spart_ref, out_ref):
    s = jnp.sum(spart_ref[...], axis=0)
    Usum = U_ref[0] + U_ref[1]
    inv = jnp.where(s > 0, 1.0 / s, 0.0)
    out_ref[...] = sp_ref[...] + Usum * inv[:, None]


def _finalize(self_pre, U, s_part):
    B = 1000
    return pl.pallas_call(
        _fin_body,
        grid=(N // B,),
        in_specs=[pl.BlockSpec((B, D), lambda i: (i, 0)),
                  pl.BlockSpec((NC, B, D), lambda i: (0, i, 0)),
                  pl.BlockSpec((NW, B), lambda i: (0, i))],
        out_specs=pl.BlockSpec((B, D), lambda i: (i, 0)),
        out_shape=jax.ShapeDtypeStruct((N, D), jnp.float32),
    )(self_pre, U, s_part)


def kernel(feat, edge_index, edge_r, attn_e, W_self, b_self, W_neigh, b_neigh):
    src = edge_index[0].astype(jnp.int32)
    dst = edge_index[1].astype(jnp.int32)
    z = _edge_z(edge_r, attn_e)
    fw, self_pre = _pre_matmuls(feat, W_neigh, W_self, b_self + b_neigh)
    U, s_part = _sc_aggregate(fw, src, dst, z)
    return _finalize(self_pre, U, s_part)


# speedup vs baseline: 11.3658x; 11.3658x over previous
import dataclasses
import functools

import jax
import jax.numpy as jnp
from jax import lax
from jax.experimental import pallas as pl
from jax.experimental.pallas import tpu as pltpu
from jax.experimental.pallas import tpu_sc as plsc

N = 10000
E = 320000
D = 128
DE = 16
NEG = 0.2

NC = 2
NS = 16
NW = NC * NS
EPW = E // NW
CE = 80
NCHUNK = EPW // CE
RPS = N // NS


def _z_body(er_ref, at_ref, sel_ref, z_ref):
    t = er_ref[...] * at_ref[...]
    l = lax.dot_general(t, sel_ref[...], (((1,), (0,)), ((), ())),
                        preferred_element_type=jnp.float32)
    l = jnp.where(l >= 0, l, NEG * l)
    z_ref[...] = jnp.exp(l)


def _edge_z(edge_r, attn_e):
    er2 = edge_r.reshape(E // 8, 8 * DE)
    at_tiled = jnp.tile(attn_e[:, 0], 8).reshape(1, 8 * DE)
    sel = jnp.kron(jnp.eye(8, dtype=jnp.float32),
                   jnp.ones((DE, 1), dtype=jnp.float32))
    B = 4000
    z8 = pl.pallas_call(
        _z_body,
        grid=(E // 8 // B,),
        in_specs=[pl.BlockSpec((B, 8 * DE), lambda i: (i, 0)),
                  pl.BlockSpec((1, 8 * DE), lambda i: (0, 0)),
                  pl.BlockSpec((8 * DE, 8), lambda i: (0, 0))],
        out_specs=pl.BlockSpec((B, 8), lambda i: (i, 0)),
        out_shape=jax.ShapeDtypeStruct((E // 8, 8), jnp.float32),
    )(er2, at_tiled, sel)
    return z8.reshape(E)


def _mm_body(f_ref, wn_ref, ws_ref, b_ref, fw_ref, sp_ref):
    f = f_ref[...]
    fw_ref[...] = lax.dot_general(f, wn_ref[...], (((1,), (1,)), ((), ())),
                                  preferred_element_type=jnp.float32)
    sp_ref[...] = lax.dot_general(f, ws_ref[...], (((1,), (1,)), ((), ())),
                                  preferred_element_type=jnp.float32) + b_ref[...]


def _pre_matmuls(feat, W_neigh, W_self, bias):
    B = 1000
    return pl.pallas_call(
        _mm_body,
        grid=(N // B,),
        in_specs=[pl.BlockSpec((B, D), lambda i: (i, 0)),
                  pl.BlockSpec((D, D), lambda i: (0, 0)),
                  pl.BlockSpec((D, D), lambda i: (0, 0)),
                  pl.BlockSpec((1, D), lambda i: (0, 0))],
        out_specs=[pl.BlockSpec((B, D), lambda i: (i, 0)),
                   pl.BlockSpec((B, D), lambda i: (i, 0))],
        out_shape=[jax.ShapeDtypeStruct((N, D), jnp.float32),
                   jax.ShapeDtypeStruct((N, D), jnp.float32)],
    )(feat, W_neigh, W_self, bias.reshape(1, D))


def _splat(vec16, j):
    idx = jnp.full((16,), j, dtype=jnp.int32)
    return lax.gather(
        vec16, idx[:, None],
        lax.GatherDimensionNumbers(offset_dims=(), collapsed_slice_dims=(0,),
                                   start_index_map=(0,)),
        slice_sizes=(1,), mode=lax.GatherScatterMode.PROMISE_IN_BOUNDS)


def _sc_body(fw_hbm, src_hbm, dst_hbm, z_hbm, U_hbm, s_hbm,
             src_v, dst_v, z_v, rows_v, s_loc, U_sh, sem):
    c = lax.axis_index("c")
    sid = lax.axis_index("s")
    wid = sid * NC + c
    zero16 = jnp.zeros((16,), jnp.float32)

    @pl.loop(0, N, step=16)
    def _(i):
        s_loc[pl.ds(i, 16)] = zero16

    @pl.loop(0, CE)
    def _(r):
        for v in range(D // 16):
            rows_v[r, pl.ds(v * 16, 16)] = zero16

    base_row = sid * 624
    for t in range(7):
        pltpu.sync_copy(rows_v.at[pl.ds(0, 80)],
                        U_sh.at[pl.ds(base_row + t * 80, 80)])

    @pl.when(sid < NS - 1)
    def _():
        pltpu.sync_copy(rows_v.at[pl.ds(0, 64)],
                        U_sh.at[pl.ds(base_row + 560, 64)])

    @pl.when(sid == NS - 1)
    def _():
        pltpu.sync_copy(rows_v.at[pl.ds(0, 80)],
                        U_sh.at[pl.ds(base_row + 560, 80)])

    plsc.subcore_barrier()

    ebase = wid * EPW

    @pl.loop(0, NCHUNK)
    def _(k):
        eoff = ebase + k * CE
        pltpu.sync_copy(src_hbm.at[pl.ds(eoff, CE)], src_v)
        pltpu.sync_copy(dst_hbm.at[pl.ds(eoff, CE)], dst_v)
        pltpu.sync_copy(z_hbm.at[pl.ds(eoff, CE)], z_v)
        pltpu.async_copy(fw_hbm.at[src_v], rows_v, sem).wait()

        @pl.loop(0, CE // 16)
        def _(g):
            z16 = z_v[pl.ds(g * 16, 16)]
            d16 = dst_v[pl.ds(g * 16, 16)]
            plsc.addupdate_scatter(s_loc, [d16], z16)
            for j in range(16):
                zj = _splat(z16, j)
                e = g * 16 + j
                for v in range(D // 16):
                    sl = pl.ds(v * 16, 16)
                    rows_v[e, sl] = rows_v[e, sl] * zj

        pltpu.sync_copy(rows_v, U_sh.at[dst_v], add=True)

    plsc.subcore_barrier()
    pltpu.sync_copy(s_loc, s_hbm.at[wid])
    for t in range(7):
        pltpu.sync_copy(U_sh.at[pl.ds(base_row + t * 80, 80)],
                        U_hbm.at[c, pl.ds(base_row + t * 80, 80)])

    @pl.when(sid < NS - 1)
    def _():
        pltpu.sync_copy(U_sh.at[pl.ds(base_row + 560, 64)],
                        U_hbm.at[c, pl.ds(base_row + 560, 64)])

    @pl.when(sid == NS - 1)
    def _():
        pltpu.sync_copy(U_sh.at[pl.ds(base_row + 560, 80)],
                        U_hbm.at[c, pl.ds(base_row + 560, 80)])


def _sc_aggregate(fw, src, dst, z):
    mesh = plsc.VectorSubcoreMesh(core_axis_name="c", subcore_axis_name="s")
    cp = pltpu.CompilerParams()
    if "needs_layout_passes" in pltpu.CompilerParams.__dataclass_fields__:
        cp = dataclasses.replace(cp, needs_layout_passes=False)
    kern = pl.kernel(
        _sc_body,
        out_type=[jax.ShapeDtypeStruct((NC, N, D), jnp.float32),
                  jax.ShapeDtypeStruct((NW, N), jnp.float32)],
        mesh=mesh,
        scratch_types=[
            pltpu.VMEM((CE,), jnp.int32),
            pltpu.VMEM((CE,), jnp.int32),
            pltpu.VMEM((CE,), jnp.float32),
            pltpu.VMEM((CE, D), jnp.float32),
            pltpu.VMEM((N,), jnp.float32),
            pltpu.VMEM_SHARED((N, D), jnp.float32),
            pltpu.SemaphoreType.DMA,
        ],
        compiler_params=cp)
    return kern(fw, src, dst, z)


def _fin_body(sp_ref, U_ref, spart_ref, out_ref):
    s = jnp.sum(spart_ref[0], axis=0)
    Usum = U_ref[0] + U_ref[1]
    inv = jnp.where(s > 0, 1.0 / s, 0.0)
    out_ref[...] = sp_ref[...] + Usum * inv[:, None]


def _finalize(self_pre, U, s_part):
    B = 1000
    s_t = s_part.reshape(NW, N // B, B).transpose(1, 0, 2)
    return pl.pallas_call(
        _fin_body,
        grid=(N // B,),
        in_specs=[pl.BlockSpec((B, D), lambda i: (i, 0)),
                  pl.BlockSpec((NC, B, D), lambda i: (0, i, 0)),
                  pl.BlockSpec((1, NW, B), lambda i: (i, 0, 0))],
        out_specs=pl.BlockSpec((B, D), lambda i: (i, 0)),
        out_shape=jax.ShapeDtypeStruct((N, D), jnp.float32),
    )(self_pre, U, s_t)


def kernel(feat, edge_index, edge_r, attn_e, W_self, b_self, W_neigh, b_neigh):
    src = edge_index[0].astype(jnp.int32)
    dst = edge_index[1].astype(jnp.int32)
    z = _edge_z(edge_r, attn_e)
    fw, self_pre = _pre_matmuls(feat, W_neigh, W_self, b_self + b_neigh)
    U, s_part = _sc_aggregate(fw, src, dst, z)
    return _finalize(self_pre, U, s_part)


# speedup vs baseline: 19.6642x; 1.7301x over previous
import dataclasses
import functools

import jax
import jax.numpy as jnp
from jax import lax
from jax.experimental import pallas as pl
from jax.experimental.pallas import tpu as pltpu
from jax.experimental.pallas import tpu_sc as plsc

N = 10000
E = 320000
D = 128
DE = 16
NEG = 0.2

NC = 2
NS = 16
NW = NC * NS
EPW = E // NW
CE = 80
NCHUNK = EPW // CE
RPS = N // NS


def _z_body(er_ref, at_ref, sel_ref, z_ref):
    t = er_ref[...] * at_ref[...]
    l = lax.dot_general(t, sel_ref[...], (((1,), (0,)), ((), ())),
                        preferred_element_type=jnp.float32)
    l = jnp.where(l >= 0, l, NEG * l)
    z_ref[...] = jnp.exp(l)


def _edge_z(edge_r, attn_e):
    er2 = edge_r.reshape(E // 8, 8 * DE)
    at_tiled = jnp.tile(attn_e[:, 0], 8).reshape(1, 8 * DE)
    sel = jnp.kron(jnp.eye(8, dtype=jnp.float32),
                   jnp.ones((DE, 1), dtype=jnp.float32))
    B = 4000
    z8 = pl.pallas_call(
        _z_body,
        grid=(E // 8 // B,),
        in_specs=[pl.BlockSpec((B, 8 * DE), lambda i: (i, 0)),
                  pl.BlockSpec((1, 8 * DE), lambda i: (0, 0)),
                  pl.BlockSpec((8 * DE, 8), lambda i: (0, 0))],
        out_specs=pl.BlockSpec((B, 8), lambda i: (i, 0)),
        out_shape=jax.ShapeDtypeStruct((E // 8, 8), jnp.float32),
    )(er2, at_tiled, sel)
    return z8.reshape(E)


def _mm_body(f_ref, wn_ref, ws_ref, b_ref, fw_ref, sp_ref):
    f = f_ref[...]
    fw_ref[...] = lax.dot_general(f, wn_ref[...], (((1,), (1,)), ((), ())),
                                  preferred_element_type=jnp.float32)
    sp_ref[...] = lax.dot_general(f, ws_ref[...], (((1,), (1,)), ((), ())),
                                  preferred_element_type=jnp.float32) + b_ref[...]


def _pre_matmuls(feat, W_neigh, W_self, bias):
    B = 1000
    return pl.pallas_call(
        _mm_body,
        grid=(N // B,),
        in_specs=[pl.BlockSpec((B, D), lambda i: (i, 0)),
                  pl.BlockSpec((D, D), lambda i: (0, 0)),
                  pl.BlockSpec((D, D), lambda i: (0, 0)),
                  pl.BlockSpec((1, D), lambda i: (0, 0))],
        out_specs=[pl.BlockSpec((B, D), lambda i: (i, 0)),
                   pl.BlockSpec((B, D), lambda i: (i, 0))],
        out_shape=[jax.ShapeDtypeStruct((N, D), jnp.float32),
                   jax.ShapeDtypeStruct((N, D), jnp.float32)],
    )(feat, W_neigh, W_self, bias.reshape(1, D))


def _splat(vec16, j):
    idx = jnp.full((16,), j, dtype=jnp.int32)
    return lax.gather(
        vec16, idx[:, None],
        lax.GatherDimensionNumbers(offset_dims=(), collapsed_slice_dims=(0,),
                                   start_index_map=(0,)),
        slice_sizes=(1,), mode=lax.GatherScatterMode.PROMISE_IN_BOUNDS)


def _sc_body(fw_hbm, src_hbm, dst_hbm, z_hbm, U_hbm, s_hbm,
             src_loc, z_loc, s_loc, rows0, rows1, db0, db1, U_sh,
             sem_st, sem_g, sem_d, sem_s0, sem_s1):
    c = lax.axis_index("c")
    sid = lax.axis_index("s")
    wid = sid * NC + c
    ebase = wid * EPW
    zero16 = jnp.zeros((16,), jnp.float32)

    st_src = pltpu.make_async_copy(src_hbm.at[pl.ds(ebase, EPW)], src_loc,
                                   sem_st)
    st_z = pltpu.make_async_copy(z_hbm.at[pl.ds(ebase, EPW)], z_loc, sem_st)
    st_src.start()
    st_z.start()

    @pl.loop(0, N, step=16)
    def _(i):
        s_loc[pl.ds(i, 16)] = zero16

    @pl.loop(0, CE)
    def _(r):
        for v in range(D // 16):
            rows0[r, pl.ds(v * 16, 16)] = zero16

    base_row = sid * 624
    for t in range(7):
        pltpu.sync_copy(rows0.at[pl.ds(0, 80)],
                        U_sh.at[pl.ds(base_row + t * 80, 80)])

    @pl.when(sid < NS - 1)
    def _():
        pltpu.sync_copy(rows0.at[pl.ds(0, 64)],
                        U_sh.at[pl.ds(base_row + 560, 64)])

    @pl.when(sid == NS - 1)
    def _():
        pltpu.sync_copy(rows0.at[pl.ds(0, 80)],
                        U_sh.at[pl.ds(base_row + 560, 80)])

    plsc.subcore_barrier()
    st_src.wait()
    st_z.wait()

    def dst_start(k, db):
        pltpu.async_copy(dst_hbm.at[pl.ds(ebase + k * CE, CE)], db, sem_d)

    def dst_wait(db):
        pltpu.make_async_copy(dst_hbm.at[pl.ds(ebase, CE)], db, sem_d).wait()

    def gather_start(k, rows):
        pltpu.async_copy(fw_hbm.at[src_loc.at[pl.ds(k * CE, CE)]], rows,
                         sem_g)

    def gather_wait(rows):
        pltpu.make_async_copy(fw_hbm.at[src_loc.at[pl.ds(0, CE)]], rows,
                              sem_g).wait()

    def scatter_start(rows, db, sem):
        pltpu.make_async_copy(rows, U_sh.at[db], sem).start(add=True)

    def scatter_wait(rows, db, sem):
        pltpu.make_async_copy(rows, U_sh.at[db], sem).wait()

    dst_start(0, db0)
    gather_start(0, rows0)

    def chunk_body(k, rows_b, db_b, sem_b, rows_o, db_o, sem_o):
        gather_wait(rows_b)
        dst_wait(db_b)

        @pl.when(k >= 1)
        def _():
            scatter_wait(rows_o, db_o, sem_o)

        @pl.when(k + 1 < NCHUNK)
        def _():
            dst_start(k + 1, db_o)
            gather_start(k + 1, rows_o)

        @pl.loop(0, CE // 16)
        def _(g):
            off = k * CE + g * 16
            z16 = z_loc[pl.ds(off, 16)]
            d16 = db_b[pl.ds(g * 16, 16)]
            plsc.addupdate_scatter(s_loc, [d16], z16)
            for j in range(16):
                zj = _splat(z16, j)
                e = g * 16 + j
                for v in range(D // 16):
                    sl = pl.ds(v * 16, 16)
                    rows_b[e, sl] = rows_b[e, sl] * zj

        scatter_start(rows_b, db_b, sem_b)

    @pl.loop(0, NCHUNK)
    def _(k):
        @pl.when(k % 2 == 0)
        def _():
            chunk_body(k, rows0, db0, sem_s0, rows1, db1, sem_s1)

        @pl.when(k % 2 == 1)
        def _():
            chunk_body(k, rows1, db1, sem_s1, rows0, db0, sem_s0)

    scatter_wait(rows0, db0, sem_s0)
    plsc.subcore_barrier()
    pltpu.sync_copy(s_loc, s_hbm.at[wid])
    for t in range(7):
        pltpu.sync_copy(U_sh.at[pl.ds(base_row + t * 80, 80)],
                        U_hbm.at[c, pl.ds(base_row + t * 80, 80)])

    @pl.when(sid < NS - 1)
    def _():
        pltpu.sync_copy(U_sh.at[pl.ds(base_row + 560, 64)],
                        U_hbm.at[c, pl.ds(base_row + 560, 64)])

    @pl.when(sid == NS - 1)
    def _():
        pltpu.sync_copy(U_sh.at[pl.ds(base_row + 560, 80)],
                        U_hbm.at[c, pl.ds(base_row + 560, 80)])


def _sc_aggregate(fw, src, dst, z):
    mesh = plsc.VectorSubcoreMesh(core_axis_name="c", subcore_axis_name="s")
    cp = pltpu.CompilerParams()
    if "needs_layout_passes" in pltpu.CompilerParams.__dataclass_fields__:
        cp = dataclasses.replace(cp, needs_layout_passes=False)
    kern = pl.kernel(
        _sc_body,
        out_type=[jax.ShapeDtypeStruct((NC, N, D), jnp.float32),
                  jax.ShapeDtypeStruct((NW, N), jnp.float32)],
        mesh=mesh,
        scratch_types=[
            pltpu.VMEM((EPW,), jnp.int32),
            pltpu.VMEM((EPW,), jnp.float32),
            pltpu.VMEM((N,), jnp.float32),
            pltpu.VMEM((CE, D), jnp.float32),
            pltpu.VMEM((CE, D), jnp.float32),
            pltpu.VMEM((CE,), jnp.int32),
            pltpu.VMEM((CE,), jnp.int32),
            pltpu.VMEM_SHARED((N, D), jnp.float32),
            pltpu.SemaphoreType.DMA,
            pltpu.SemaphoreType.DMA,
            pltpu.SemaphoreType.DMA,
            pltpu.SemaphoreType.DMA,
            pltpu.SemaphoreType.DMA,
        ],
        compiler_params=cp)
    return kern(fw, src, dst, z)


def _fin_body(sp_ref, U_ref, spart_ref, out_ref):
    s = jnp.sum(spart_ref[0], axis=0)
    Usum = U_ref[0] + U_ref[1]
    inv = jnp.where(s > 0, 1.0 / s, 0.0)
    out_ref[...] = sp_ref[...] + Usum * inv[:, None]


def _finalize(self_pre, U, s_part):
    B = 1000
    s_t = s_part.reshape(NW, N // B, B).transpose(1, 0, 2)
    return pl.pallas_call(
        _fin_body,
        grid=(N // B,),
        in_specs=[pl.BlockSpec((B, D), lambda i: (i, 0)),
                  pl.BlockSpec((NC, B, D), lambda i: (0, i, 0)),
                  pl.BlockSpec((1, NW, B), lambda i: (i, 0, 0))],
        out_specs=pl.BlockSpec((B, D), lambda i: (i, 0)),
        out_shape=jax.ShapeDtypeStruct((N, D), jnp.float32),
    )(self_pre, U, s_t)


def kernel(feat, edge_index, edge_r, attn_e, W_self, b_self, W_neigh, b_neigh):
    src = edge_index[0].astype(jnp.int32)
    dst = edge_index[1].astype(jnp.int32)
    z = _edge_z(edge_r, attn_e)
    fw, self_pre = _pre_matmuls(feat, W_neigh, W_self, b_self + b_neigh)
    U, s_part = _sc_aggregate(fw, src, dst, z)
    return _finalize(self_pre, U, s_part)


# speedup vs baseline: 19.8208x; 1.0080x over previous
import dataclasses
import functools

import jax
import jax.numpy as jnp
from jax import lax
from jax.experimental import pallas as pl
from jax.experimental.pallas import tpu as pltpu
from jax.experimental.pallas import tpu_sc as plsc

N = 10000
E = 320000
D = 128
DE = 16
NEG = 0.2

NC = 2
NS = 16
NW = NC * NS
EPW = E // NW
CE = 80
NCHUNK = EPW // CE
RPS = N // NS


def _pre_body(er_ref, at_ref, sel_ref, f_ref, wn_ref, ws_ref, b_ref,
              z_ref, fw_ref, sp_ref):
    t = er_ref[...] * at_ref[...]
    l = lax.dot_general(t, sel_ref[...], (((1,), (0,)), ((), ())),
                        preferred_element_type=jnp.float32)
    l = jnp.where(l >= 0, l, NEG * l)
    z_ref[...] = jnp.exp(l)
    f = f_ref[...]
    fw_ref[...] = lax.dot_general(f, wn_ref[...], (((1,), (1,)), ((), ())),
                                  preferred_element_type=jnp.float32)
    sp_ref[...] = lax.dot_general(f, ws_ref[...], (((1,), (1,)), ((), ())),
                                  preferred_element_type=jnp.float32) + b_ref[...]


def _prologue(edge_r, attn_e, feat, W_neigh, W_self, bias):
    er2 = edge_r.reshape(E // 8, 8 * DE)
    at_tiled = jnp.tile(attn_e[:, 0], 8).reshape(1, 8 * DE)
    sel = jnp.kron(jnp.eye(8, dtype=jnp.float32),
                   jnp.ones((DE, 1), dtype=jnp.float32))
    G = 10
    BE = E // 8 // G
    BN = N // G
    z8, fw, sp = pl.pallas_call(
        _pre_body,
        grid=(G,),
        in_specs=[pl.BlockSpec((BE, 8 * DE), lambda i: (i, 0)),
                  pl.BlockSpec((1, 8 * DE), lambda i: (0, 0)),
                  pl.BlockSpec((8 * DE, 8), lambda i: (0, 0)),
                  pl.BlockSpec((BN, D), lambda i: (i, 0)),
                  pl.BlockSpec((D, D), lambda i: (0, 0)),
                  pl.BlockSpec((D, D), lambda i: (0, 0)),
                  pl.BlockSpec((1, D), lambda i: (0, 0))],
        out_specs=[pl.BlockSpec((BE, 8), lambda i: (i, 0)),
                   pl.BlockSpec((BN, D), lambda i: (i, 0)),
                   pl.BlockSpec((BN, D), lambda i: (i, 0))],
        out_shape=[jax.ShapeDtypeStruct((E // 8, 8), jnp.float32),
                   jax.ShapeDtypeStruct((N, D), jnp.float32),
                   jax.ShapeDtypeStruct((N, D), jnp.float32)],
    )(er2, at_tiled, sel, feat, W_neigh, W_self, bias.reshape(1, D))
    return z8.reshape(E), fw, sp


def _splat(vec16, j):
    idx = jnp.full((16,), j, dtype=jnp.int32)
    return lax.gather(
        vec16, idx[:, None],
        lax.GatherDimensionNumbers(offset_dims=(), collapsed_slice_dims=(0,),
                                   start_index_map=(0,)),
        slice_sizes=(1,), mode=lax.GatherScatterMode.PROMISE_IN_BOUNDS)


def _sc_body(fw_hbm, ei_hbm, z_hbm, U_hbm, s_hbm,
             src_loc, z_loc, s_loc, rows0, rows1, db0, db1, U_sh,
             sem_st, sem_g, sem_d, sem_s0, sem_s1):
    c = lax.axis_index("c")
    sid = lax.axis_index("s")
    wid = sid * NC + c
    ebase = wid * EPW
    zero16 = jnp.zeros((16,), jnp.float32)

    st_src = pltpu.make_async_copy(ei_hbm.at[pl.ds(ebase, EPW)], src_loc,
                                   sem_st)
    st_z = pltpu.make_async_copy(z_hbm.at[pl.ds(ebase, EPW)], z_loc, sem_st)
    st_src.start()
    st_z.start()

    @pl.loop(0, N, step=16)
    def _(i):
        s_loc[pl.ds(i, 16)] = zero16

    @pl.loop(0, CE)
    def _(r):
        for v in range(D // 16):
            rows0[r, pl.ds(v * 16, 16)] = zero16

    base_row = sid * 624
    for t in range(7):
        pltpu.sync_copy(rows0.at[pl.ds(0, 80)],
                        U_sh.at[pl.ds(base_row + t * 80, 80)])

    @pl.when(sid < NS - 1)
    def _():
        pltpu.sync_copy(rows0.at[pl.ds(0, 64)],
                        U_sh.at[pl.ds(base_row + 560, 64)])

    @pl.when(sid == NS - 1)
    def _():
        pltpu.sync_copy(rows0.at[pl.ds(0, 80)],
                        U_sh.at[pl.ds(base_row + 560, 80)])

    plsc.subcore_barrier()
    st_src.wait()
    st_z.wait()

    def dst_start(k, db):
        pltpu.async_copy(ei_hbm.at[pl.ds(E + ebase + k * CE, CE)], db, sem_d)

    def dst_wait(db):
        pltpu.make_async_copy(ei_hbm.at[pl.ds(E, CE)], db, sem_d).wait()

    def gather_start(k, rows):
        pltpu.async_copy(fw_hbm.at[src_loc.at[pl.ds(k * CE, CE)]], rows,
                         sem_g)

    def gather_wait(rows):
        pltpu.make_async_copy(fw_hbm.at[src_loc.at[pl.ds(0, CE)]], rows,
                              sem_g).wait()

    def scatter_start(rows, db, sem):
        pltpu.make_async_copy(rows, U_sh.at[db], sem).start(add=True)

    def scatter_wait(rows, db, sem):
        pltpu.make_async_copy(rows, U_sh.at[db], sem).wait()

    dst_start(0, db0)
    gather_start(0, rows0)

    def chunk_body(k, rows_b, db_b, sem_b, rows_o, db_o, sem_o):
        gather_wait(rows_b)
        dst_wait(db_b)

        @pl.when(k >= 1)
        def _():
            scatter_wait(rows_o, db_o, sem_o)

        @pl.when(k + 1 < NCHUNK)
        def _():
            dst_start(k + 1, db_o)
            gather_start(k + 1, rows_o)

        @pl.loop(0, CE // 16)
        def _(g):
            off = k * CE + g * 16
            z16 = z_loc[pl.ds(off, 16)]
            d16 = db_b[pl.ds(g * 16, 16)]
            plsc.addupdate_scatter(s_loc, [d16], z16)
            for j in range(16):
                zj = _splat(z16, j)
                e = g * 16 + j
                for v in range(D // 16):
                    sl = pl.ds(v * 16, 16)
                    rows_b[e, sl] = rows_b[e, sl] * zj

        scatter_start(rows_b, db_b, sem_b)

    @pl.loop(0, NCHUNK)
    def _(k):
        @pl.when(k % 2 == 0)
        def _():
            chunk_body(k, rows0, db0, sem_s0, rows1, db1, sem_s1)

        @pl.when(k % 2 == 1)
        def _():
            chunk_body(k, rows1, db1, sem_s1, rows0, db0, sem_s0)

    scatter_wait(rows0, db0, sem_s0)
    plsc.subcore_barrier()
    pltpu.sync_copy(s_loc, s_hbm.at[wid])
    for t in range(7):
        pltpu.sync_copy(U_sh.at[pl.ds(base_row + t * 80, 80)],
                        U_hbm.at[c, pl.ds(base_row + t * 80, 80)])

    @pl.when(sid < NS - 1)
    def _():
        pltpu.sync_copy(U_sh.at[pl.ds(base_row + 560, 64)],
                        U_hbm.at[c, pl.ds(base_row + 560, 64)])

    @pl.when(sid == NS - 1)
    def _():
        pltpu.sync_copy(U_sh.at[pl.ds(base_row + 560, 80)],
                        U_hbm.at[c, pl.ds(base_row + 560, 80)])


def _sc_aggregate(fw, ei_flat, z):
    mesh = plsc.VectorSubcoreMesh(core_axis_name="c", subcore_axis_name="s")
    cp = pltpu.CompilerParams()
    if "needs_layout_passes" in pltpu.CompilerParams.__dataclass_fields__:
        cp = dataclasses.replace(cp, needs_layout_passes=False)
    kern = pl.kernel(
        _sc_body,
        out_type=[jax.ShapeDtypeStruct((NC, N, D), jnp.float32),
                  jax.ShapeDtypeStruct((NW, N), jnp.float32)],
        mesh=mesh,
        scratch_types=[
            pltpu.VMEM((EPW,), jnp.int32),
            pltpu.VMEM((EPW,), jnp.float32),
            pltpu.VMEM((N,), jnp.float32),
            pltpu.VMEM((CE, D), jnp.float32),
            pltpu.VMEM((CE, D), jnp.float32),
            pltpu.VMEM((CE,), jnp.int32),
            pltpu.VMEM((CE,), jnp.int32),
            pltpu.VMEM_SHARED((N, D), jnp.float32),
            pltpu.SemaphoreType.DMA,
            pltpu.SemaphoreType.DMA,
            pltpu.SemaphoreType.DMA,
            pltpu.SemaphoreType.DMA,
            pltpu.SemaphoreType.DMA,
        ],
        compiler_params=cp)
    return kern(fw, ei_flat, z)


def _fin_body(sp_ref, U_ref, spart_ref, out_ref):
    s = jnp.sum(spart_ref[0], axis=0)
    Usum = U_ref[0] + U_ref[1]
    inv = jnp.where(s > 0, 1.0 / s, 0.0)
    out_ref[...] = sp_ref[...] + Usum * inv[:, None]


def _finalize(self_pre, U, s_part):
    B = N // 10
    s_t = s_part.reshape(NW, N // B, B).transpose(1, 0, 2)
    return pl.pallas_call(
        _fin_body,
        grid=(N // B,),
        in_specs=[pl.BlockSpec((B, D), lambda i: (i, 0)),
                  pl.BlockSpec((NC, B, D), lambda i: (0, i, 0)),
                  pl.BlockSpec((1, NW, B), lambda i: (i, 0, 0))],
        out_specs=pl.BlockSpec((B, D), lambda i: (i, 0)),
        out_shape=jax.ShapeDtypeStruct((N, D), jnp.float32),
    )(self_pre, U, s_t)


def kernel(feat, edge_index, edge_r, attn_e, W_self, b_self, W_neigh, b_neigh):
    ei_flat = edge_index.astype(jnp.int32).reshape(2 * E)
    z, fw, self_pre = _prologue(edge_r, attn_e, feat, W_neigh, W_self,
                                b_self + b_neigh)
    U, s_part = _sc_aggregate(fw, ei_flat, z)
    return _finalize(self_pre, U, s_part)
